# Initial kernel scaffold; baseline (speedup 1.0000x reference)
import jax
import jax.numpy as jnp
from jax.experimental import pallas as pl


def kernel(obj_fmaps, obj_logits, vr, rel_inds, global_features, W_proj, b_proj, W_vr, b_vr):
    raise NotImplementedError("write your pallas kernel here")


# speedup vs baseline: 4.0832x; 4.0832x over previous
import functools

import jax
import jax.numpy as jnp
from jax import lax
from jax.experimental import pallas as pl
from jax.experimental.pallas import tpu as pltpu
from jax.experimental.pallas import tpu_sc as plsc

N_OBJ = 5000
N_REL = 20000
D_OBJ = 1024
REL_DIM = 512
REL_OUT = 64
NUM_OBJ_CLS = 151
NCLS = 51
CP = 128
ADD_COLS = 64

NW = 32
NREL_PAD = 20480
PER_W = NREL_PAD // NW
CH = 128
NCH = PER_W // CH


def _tables_body(of_ref, gf_ref, wab_ref, wg_ref, a_ref, b_ref, g_ref):
    x = of_ref[...]
    ab = jnp.dot(x, wab_ref[...], preferred_element_type=jnp.float32)
    a_ref[...] = ab[:, :CP]
    b_ref[...] = ab[:, CP:]
    g_ref[...] = jnp.dot(gf_ref[...], wg_ref[...],
                         preferred_element_type=jnp.float32)


def _make_tables(obj_fmaps, global_features, w_ab, w_g):
    blk = 1000
    grid = N_OBJ // blk
    return pl.pallas_call(
        _tables_body,
        grid=(grid,),
        in_specs=[
            pl.BlockSpec((blk, D_OBJ), lambda i: (i, 0)),
            pl.BlockSpec((blk, D_OBJ), lambda i: (i, 0)),
            pl.BlockSpec((D_OBJ, 2 * CP), lambda i: (0, 0)),
            pl.BlockSpec((D_OBJ, CP), lambda i: (0, 0)),
        ],
        out_specs=[
            pl.BlockSpec((blk, CP), lambda i: (i, 0)),
            pl.BlockSpec((blk, CP), lambda i: (i, 0)),
            pl.BlockSpec((blk, CP), lambda i: (i, 0)),
        ],
        out_shape=[
            jax.ShapeDtypeStruct((N_OBJ, CP), jnp.float32),
            jax.ShapeDtypeStruct((N_OBJ, CP), jnp.float32),
            jax.ShapeDtypeStruct((N_OBJ, CP), jnp.float32),
        ],
    )(obj_fmaps, global_features, w_ab, w_g)


def _vrterm_body(vr_ref, wpt_ref, bp_ref, wv_ref, bv_ref, r_ref):
    vp = jnp.dot(vr_ref[...], wpt_ref[...],
                 preferred_element_type=jnp.float32) + bp_ref[...]
    r_ref[...] = jnp.dot(vp, wv_ref[...],
                         preferred_element_type=jnp.float32) + bv_ref[...]


def _make_vrterm(vr, wpt, bp, wv, bv):
    blk = 2000
    grid = N_REL // blk
    return pl.pallas_call(
        _vrterm_body,
        grid=(grid,),
        in_specs=[
            pl.BlockSpec((blk, REL_DIM), lambda i: (i, 0)),
            pl.BlockSpec((REL_DIM, REL_OUT), lambda i: (0, 0)),
            pl.BlockSpec((1, REL_OUT), lambda i: (0, 0)),
            pl.BlockSpec((REL_OUT, CP), lambda i: (0, 0)),
            pl.BlockSpec((1, CP), lambda i: (0, 0)),
        ],
        out_specs=pl.BlockSpec((blk, CP), lambda i: (i, 0)),
        out_shape=jax.ShapeDtypeStruct((N_REL, CP), jnp.float32),
    )(vr, wpt, bp, wv, bv)


def _argmax_body(lg_ref, out_ref):
    x = lg_ref[...]
    col = lax.broadcasted_iota(jnp.int32, x.shape, 1)
    xm = jnp.where(col >= 1, x, -jnp.inf)
    out_ref[...] = jnp.argmax(xm, axis=1).astype(jnp.int32)[:, None]


def _make_argmax(obj_logits):
    blk = 1000
    grid = N_OBJ // blk
    return pl.pallas_call(
        _argmax_body,
        grid=(grid,),
        in_specs=[pl.BlockSpec((blk, NUM_OBJ_CLS), lambda i: (i, 0))],
        out_specs=pl.BlockSpec((blk, 1), lambda i: (i, 0)),
        out_shape=jax.ShapeDtypeStruct((N_OBJ, 1), jnp.int32),
    )(obj_logits)


def _sc_body(a_hbm, b_hbm, g_hbm, r_hbm, ig_hbm, is_hbm, io_hbm, out_hbm,
             ig_v, is_v, io_v, acc_v, ra_v, rb_v, rg_v, sem_a, sem_b, sem_g):
    wid = lax.axis_index("s") * 2 + lax.axis_index("c")
    base = wid * PER_W
    pltpu.sync_copy(ig_hbm.at[pl.ds(base, PER_W)], ig_v)
    pltpu.sync_copy(is_hbm.at[pl.ds(base, PER_W)], is_v)
    pltpu.sync_copy(io_hbm.at[pl.ds(base, PER_W)], io_v)

    def chunk(k, carry):
        off = k * CH
        cpa = pltpu.async_copy(a_hbm.at[is_v.at[pl.ds(off, CH)]], ra_v, sem_a)
        cpb = pltpu.async_copy(b_hbm.at[io_v.at[pl.ds(off, CH)]], rb_v, sem_b)
        cpg = pltpu.async_copy(g_hbm.at[ig_v.at[pl.ds(off, CH)]], rg_v, sem_g)
        pltpu.sync_copy(r_hbm.at[pl.ds(base + off, CH)], acc_v)
        cpa.wait()
        cpb.wait()
        cpg.wait()

        def row(i, c2):
            for j in range(ADD_COLS // 16):
                sl = pl.ds(j * 16, 16)
                acc_v[i, sl] = acc_v[i, sl] + ra_v[i, sl] + rb_v[i, sl] \
                    + rg_v[i, sl]
            return c2

        lax.fori_loop(0, CH, row, 0)
        pltpu.sync_copy(acc_v, out_hbm.at[pl.ds(base + off, CH)])
        return carry

    lax.fori_loop(0, NCH, chunk, 0)


def _sc_gather_combine(a_t, b_t, g_t, r_t, idx_img, idx_subj, idx_obj):
    mesh = plsc.VectorSubcoreMesh(core_axis_name="c", subcore_axis_name="s")
    k = functools.partial(
        pl.kernel,
        mesh=mesh,
        out_type=jax.ShapeDtypeStruct((NREL_PAD, CP), jnp.float32),
        scratch_types=[
            pltpu.VMEM((PER_W,), jnp.int32),
            pltpu.VMEM((PER_W,), jnp.int32),
            pltpu.VMEM((PER_W,), jnp.int32),
            pltpu.VMEM((CH, CP), jnp.float32),
            pltpu.VMEM((CH, CP), jnp.float32),
            pltpu.VMEM((CH, CP), jnp.float32),
            pltpu.VMEM((CH, CP), jnp.float32),
            pltpu.SemaphoreType.DMA,
            pltpu.SemaphoreType.DMA,
            pltpu.SemaphoreType.DMA,
        ],
    )(_sc_body)
    return k(a_t, b_t, g_t, r_t, idx_img, idx_subj, idx_obj)


def kernel(obj_fmaps, obj_logits, vr, rel_inds, global_features,
           W_proj, b_proj, W_vr, b_vr):
    wt = W_vr.T
    pad_c = lambda w: jnp.pad(w, ((0, 0), (0, CP - NCLS)))
    w1t = pad_c(wt[:D_OBJ])
    w2t = pad_c(wt[D_OBJ:2 * D_OBJ])
    wv = pad_c(wt[2 * D_OBJ:2 * D_OBJ + REL_OUT])
    wg = pad_c(wt[2 * D_OBJ + REL_OUT:])
    w_ab = jnp.concatenate([w1t, w2t], axis=1)
    wpt = W_proj.T
    bp = b_proj[None, :]
    bv = pad_c(b_vr[None, :])

    a_t, b_t, g_t = _make_tables(obj_fmaps, global_features, w_ab, wg)
    r_t = _make_vrterm(vr, wpt, bp, wv, bv)
    preds = _make_argmax(obj_logits)

    pad_n = NREL_PAD - N_REL
    idx_img = jnp.pad(rel_inds[:, 0], (0, pad_n))
    idx_subj = jnp.pad(rel_inds[:, 1], (0, pad_n))
    idx_obj = jnp.pad(rel_inds[:, 2], (0, pad_n))
    r_pad = jnp.pad(r_t, ((0, pad_n), (0, 0)))
    out = _sc_gather_combine(a_t, b_t, g_t, r_pad, idx_img, idx_subj, idx_obj)

    rel_dists = out[:N_REL, :NCLS]
    return (obj_logits, preds.reshape(-1), rel_dists)


# speedup vs baseline: 4.3697x; 1.0702x over previous
import functools

import jax
import jax.numpy as jnp
from jax import lax
from jax.experimental import pallas as pl
from jax.experimental.pallas import tpu as pltpu
from jax.experimental.pallas import tpu_sc as plsc

N_OBJ = 5000
N_REL = 20000
D_OBJ = 1024
REL_DIM = 512
REL_OUT = 64
NUM_OBJ_CLS = 151
NCLS = 51
CP = 128

NW = 32
NREL_PAD = 20480
PER_W = NREL_PAD // NW
CH = 128
NCH = PER_W // CH


def _tables_body(of_ref, gf_ref, wab_ref, wg_ref, a_ref, b_ref, g_ref):
    x = of_ref[...]
    ab = jnp.dot(x, wab_ref[...], preferred_element_type=jnp.float32)
    a_ref[...] = ab[:, :CP]
    b_ref[...] = ab[:, CP:]
    g_ref[...] = jnp.dot(gf_ref[...], wg_ref[...],
                         preferred_element_type=jnp.float32)


def _make_tables(obj_fmaps, global_features, w_ab, w_g):
    blk = 1000
    grid = N_OBJ // blk
    return pl.pallas_call(
        _tables_body,
        grid=(grid,),
        in_specs=[
            pl.BlockSpec((blk, D_OBJ), lambda i: (i, 0)),
            pl.BlockSpec((blk, D_OBJ), lambda i: (i, 0)),
            pl.BlockSpec((D_OBJ, 2 * CP), lambda i: (0, 0)),
            pl.BlockSpec((D_OBJ, CP), lambda i: (0, 0)),
        ],
        out_specs=[
            pl.BlockSpec((blk, CP), lambda i: (i, 0)),
            pl.BlockSpec((blk, CP), lambda i: (i, 0)),
            pl.BlockSpec((blk, CP), lambda i: (i, 0)),
        ],
        out_shape=[
            jax.ShapeDtypeStruct((N_OBJ, CP), jnp.float32),
            jax.ShapeDtypeStruct((N_OBJ, CP), jnp.float32),
            jax.ShapeDtypeStruct((N_OBJ, CP), jnp.float32),
        ],
    )(obj_fmaps, global_features, w_ab, w_g)


def _vrterm_body(vr_ref, wpt_ref, bp_ref, wv_ref, bv_ref, r_ref):
    vp = jnp.dot(vr_ref[...], wpt_ref[...],
                 preferred_element_type=jnp.float32) + bp_ref[...]
    r_ref[...] = jnp.dot(vp, wv_ref[...],
                         preferred_element_type=jnp.float32) + bv_ref[...]


def _make_vrterm(vr, wpt, bp, wv, bv):
    blk = 1280
    grid = NREL_PAD // blk
    return pl.pallas_call(
        _vrterm_body,
        grid=(grid,),
        in_specs=[
            pl.BlockSpec((blk, REL_DIM), lambda i: (i, 0)),
            pl.BlockSpec((REL_DIM, REL_OUT), lambda i: (0, 0)),
            pl.BlockSpec((1, REL_OUT), lambda i: (0, 0)),
            pl.BlockSpec((REL_OUT, CP), lambda i: (0, 0)),
            pl.BlockSpec((1, CP), lambda i: (0, 0)),
        ],
        out_specs=pl.BlockSpec((blk, CP), lambda i: (i, 0)),
        out_shape=jax.ShapeDtypeStruct((NREL_PAD, CP), jnp.float32),
    )(vr, wpt, bp, wv, bv)


def _argmax_body(lg_ref, out_ref):
    x = lg_ref[...]
    col = lax.broadcasted_iota(jnp.int32, x.shape, 1)
    xm = jnp.where(col >= 1, x, -jnp.inf)
    out_ref[...] = jnp.argmax(xm, axis=1).astype(jnp.int32)[:, None]


def _make_argmax(obj_logits):
    blk = 1000
    grid = N_OBJ // blk
    return pl.pallas_call(
        _argmax_body,
        grid=(grid,),
        in_specs=[pl.BlockSpec((blk, NUM_OBJ_CLS), lambda i: (i, 0))],
        out_specs=pl.BlockSpec((blk, 1), lambda i: (i, 0)),
        out_shape=jax.ShapeDtypeStruct((N_OBJ, 1), jnp.int32),
    )(obj_logits)


def _sc_body(a_hbm, b_hbm, g_hbm, r_hbm, ig_hbm, is_hbm, io_hbm, out_hbm,
             ig_v, is_v, io_v, acc0, acc1, sem_a, sem_b, sem_g,
             sem_o0, sem_o1):
    wid = lax.axis_index("s") * 2 + lax.axis_index("c")
    base = wid * PER_W
    pltpu.sync_copy(ig_hbm.at[pl.ds(base, PER_W)], ig_v)
    pltpu.sync_copy(is_hbm.at[pl.ds(base, PER_W)], is_v)
    pltpu.sync_copy(io_hbm.at[pl.ds(base, PER_W)], io_v)

    accs = (acc0, acc1)
    osems = (sem_o0, sem_o1)
    out_cp = [None, None]
    pltpu.sync_copy(r_hbm.at[pl.ds(base, CH)], acc0)
    for k in range(NCH):
        p = k % 2
        acc = accs[p]
        off = k * CH
        cpa = pltpu.async_copy(a_hbm.at[is_v.at[pl.ds(off, CH)]], acc,
                               sem_a, add=True)
        cpb = pltpu.async_copy(b_hbm.at[io_v.at[pl.ds(off, CH)]], acc,
                               sem_b, add=True)
        cpg = pltpu.async_copy(g_hbm.at[ig_v.at[pl.ds(off, CH)]], acc,
                               sem_g, add=True)
        if k + 1 < NCH:
            if out_cp[1 - p] is not None:
                out_cp[1 - p].wait()
            pltpu.sync_copy(r_hbm.at[pl.ds(base + off + CH, CH)],
                            accs[1 - p])
        cpa.wait()
        cpb.wait()
        cpg.wait()
        out_cp[p] = pltpu.async_copy(acc, out_hbm.at[pl.ds(base + off, CH)],
                                     osems[p])
    out_cp[(NCH - 1) % 2].wait()
    if out_cp[NCH % 2] is not None:
        out_cp[NCH % 2].wait()


def _sc_gather_combine(a_t, b_t, g_t, r_t, idx_img, idx_subj, idx_obj):
    mesh = plsc.VectorSubcoreMesh(core_axis_name="c", subcore_axis_name="s")
    k = functools.partial(
        pl.kernel,
        mesh=mesh,
        out_type=jax.ShapeDtypeStruct((NREL_PAD, CP), jnp.float32),
        scratch_types=[
            pltpu.VMEM((PER_W,), jnp.int32),
            pltpu.VMEM((PER_W,), jnp.int32),
            pltpu.VMEM((PER_W,), jnp.int32),
            pltpu.VMEM((CH, CP), jnp.float32),
            pltpu.VMEM((CH, CP), jnp.float32),
            pltpu.SemaphoreType.DMA,
            pltpu.SemaphoreType.DMA,
            pltpu.SemaphoreType.DMA,
            pltpu.SemaphoreType.DMA,
            pltpu.SemaphoreType.DMA,
        ],
    )(_sc_body)
    return k(a_t, b_t, g_t, r_t, idx_img, idx_subj, idx_obj)


def kernel(obj_fmaps, obj_logits, vr, rel_inds, global_features,
           W_proj, b_proj, W_vr, b_vr):
    wt = W_vr.T
    pad_c = lambda w: jnp.pad(w, ((0, 0), (0, CP - NCLS)))
    w1t = pad_c(wt[:D_OBJ])
    w2t = pad_c(wt[D_OBJ:2 * D_OBJ])
    wv = pad_c(wt[2 * D_OBJ:2 * D_OBJ + REL_OUT])
    wg = pad_c(wt[2 * D_OBJ + REL_OUT:])
    w_ab = jnp.concatenate([w1t, w2t], axis=1)
    wpt = W_proj.T
    bp = b_proj[None, :]
    bv = pad_c(b_vr[None, :])

    a_t, b_t, g_t = _make_tables(obj_fmaps, global_features, w_ab, wg)
    r_t = _make_vrterm(vr, wpt, bp, wv, bv)
    preds = _make_argmax(obj_logits)

    pad_n = NREL_PAD - N_REL
    idx_img = jnp.pad(rel_inds[:, 0], (0, pad_n))
    idx_subj = jnp.pad(rel_inds[:, 1], (0, pad_n))
    idx_obj = jnp.pad(rel_inds[:, 2], (0, pad_n))
    out = _sc_gather_combine(a_t, b_t, g_t, r_t, idx_img, idx_subj, idx_obj)

    rel_dists = out[:N_REL, :NCLS]
    return (obj_logits, preds.reshape(-1), rel_dists)


# speedup vs baseline: 4.7600x; 1.0893x over previous
import functools

import jax
import jax.numpy as jnp
from jax import lax
from jax.experimental import pallas as pl
from jax.experimental.pallas import tpu as pltpu
from jax.experimental.pallas import tpu_sc as plsc

N_OBJ = 5000
N_REL = 20000
D_OBJ = 1024
REL_DIM = 512
REL_OUT = 64
NUM_OBJ_CLS = 151
NCLS = 51
CP = 128
VCP = 64

NREL_PAD = 20480
PER_W = NREL_PAD // 32
CH = 128
NCH = PER_W // CH


def _tables_body(of_ref, gf_ref, wab_ref, wg_ref, a_ref, b_ref, g_ref):
    x = of_ref[...]
    ab = jnp.dot(x, wab_ref[...], preferred_element_type=jnp.float32)
    a_ref[...] = ab[:, :CP]
    b_ref[...] = ab[:, CP:]
    g_ref[...] = jnp.dot(gf_ref[...], wg_ref[...],
                         preferred_element_type=jnp.float32)


def _make_tables(obj_fmaps, global_features, w_ab, w_g):
    blk = 1000
    grid = N_OBJ // blk
    return pl.pallas_call(
        _tables_body,
        grid=(grid,),
        in_specs=[
            pl.BlockSpec((blk, D_OBJ), lambda i: (i, 0)),
            pl.BlockSpec((blk, D_OBJ), lambda i: (i, 0)),
            pl.BlockSpec((D_OBJ, 2 * CP), lambda i: (0, 0)),
            pl.BlockSpec((D_OBJ, CP), lambda i: (0, 0)),
        ],
        out_specs=[
            pl.BlockSpec((blk, CP), lambda i: (i, 0)),
            pl.BlockSpec((blk, CP), lambda i: (i, 0)),
            pl.BlockSpec((blk, CP), lambda i: (i, 0)),
        ],
        out_shape=[
            jax.ShapeDtypeStruct((N_OBJ, CP), jnp.float32),
            jax.ShapeDtypeStruct((N_OBJ, CP), jnp.float32),
            jax.ShapeDtypeStruct((N_OBJ, CP), jnp.float32),
        ],
    )(obj_fmaps, global_features, w_ab, w_g)


def _vrterm_body(vr_ref, wpt_ref, bp_ref, wv_ref, bv_ref, r_ref):
    vp = jnp.dot(vr_ref[...], wpt_ref[...],
                 preferred_element_type=jnp.float32) + bp_ref[...]
    r_ref[...] = jnp.dot(vp, wv_ref[...],
                         preferred_element_type=jnp.float32) + bv_ref[...]


def _make_vrterm(vr, wpt, bp, wv, bv):
    blk = 2560
    grid = NREL_PAD // blk
    return pl.pallas_call(
        _vrterm_body,
        grid=(grid,),
        in_specs=[
            pl.BlockSpec((blk, REL_DIM), lambda i: (i, 0)),
            pl.BlockSpec((REL_DIM, REL_OUT), lambda i: (0, 0)),
            pl.BlockSpec((1, REL_OUT), lambda i: (0, 0)),
            pl.BlockSpec((REL_OUT, VCP), lambda i: (0, 0)),
            pl.BlockSpec((1, VCP), lambda i: (0, 0)),
        ],
        out_specs=pl.BlockSpec((blk, VCP), lambda i: (i, 0)),
        out_shape=jax.ShapeDtypeStruct((NREL_PAD, VCP), jnp.float32),
    )(vr, wpt, bp, wv, bv)


def _argmax_body(lg_ref, out_ref):
    x = lg_ref[...]
    col = lax.broadcasted_iota(jnp.int32, x.shape, 1)
    xm = jnp.where(col >= 1, x, -jnp.inf)
    out_ref[...] = jnp.argmax(xm, axis=1).astype(jnp.int32)[:, None]


def _make_argmax(obj_logits):
    blk = 1000
    grid = N_OBJ // blk
    return pl.pallas_call(
        _argmax_body,
        grid=(grid,),
        in_specs=[pl.BlockSpec((blk, NUM_OBJ_CLS), lambda i: (i, 0))],
        out_specs=pl.BlockSpec((blk, 1), lambda i: (i, 0)),
        out_shape=jax.ShapeDtypeStruct((N_OBJ, 1), jnp.int32),
    )(obj_logits)


def _combine_body(s_ref, r_ref, out_ref):
    out_ref[...] = s_ref[:, :NCLS] + r_ref[:, :NCLS]


def _make_combine(s, r):
    blk = 2000
    grid = N_REL // blk
    return pl.pallas_call(
        _combine_body,
        grid=(grid,),
        in_specs=[
            pl.BlockSpec((blk, CP), lambda i: (i, 0)),
            pl.BlockSpec((blk, VCP), lambda i: (i, 0)),
        ],
        out_specs=pl.BlockSpec((blk, NCLS), lambda i: (i, 0)),
        out_shape=jax.ShapeDtypeStruct((N_REL, NCLS), jnp.float32),
    )(s, r)


def _sc_body(a_hbm, b_hbm, g_hbm, ig_hbm, is_hbm, io_hbm, out_hbm,
             ig_v, is_v, io_v, acc0, acc1,
             sem_i, sem_a0, sem_a1, sem_bg0, sem_bg1,
             sem_o0, sem_o1):
    cid = lax.axis_index("c")
    sid = lax.axis_index("s")
    wid = sid * 2 + cid
    base = wid * PER_W
    i1 = pltpu.async_copy(ig_hbm.at[pl.ds(base, PER_W)], ig_v, sem_i)
    i2 = pltpu.async_copy(is_hbm.at[pl.ds(base, PER_W)], is_v, sem_i)
    i3 = pltpu.async_copy(io_hbm.at[pl.ds(base, PER_W)], io_v, sem_i)
    i1.wait()
    i2.wait()
    i3.wait()

    accs = (acc0, acc1)
    asems = (sem_a0, sem_a1)
    bgsems = (sem_bg0, sem_bg1)
    osems = (sem_o0, sem_o1)

    def issue_a(k):
        return pltpu.async_copy(a_hbm.at[is_v.at[pl.ds(k * CH, CH)]],
                                accs[k % 2], asems[k % 2])

    a_cp = {0: issue_a(0)}
    out_cp = [None, None]
    for k in range(NCH):
        p = k % 2
        off = k * CH
        a_cp.pop(k).wait()
        cb = pltpu.async_copy(b_hbm.at[io_v.at[pl.ds(off, CH)]], accs[p],
                              bgsems[p], add=True)
        cg = pltpu.async_copy(g_hbm.at[ig_v.at[pl.ds(off, CH)]], accs[p],
                              bgsems[p], add=True)
        if k + 1 < NCH:
            if out_cp[1 - p] is not None:
                out_cp[1 - p].wait()
            a_cp[k + 1] = issue_a(k + 1)
        cb.wait()
        cg.wait()
        out_cp[p] = pltpu.async_copy(accs[p],
                                     out_hbm.at[pl.ds(base + off, CH)],
                                     osems[p])
    out_cp[(NCH - 1) % 2].wait()
    if out_cp[NCH % 2] is not None:
        out_cp[NCH % 2].wait()


def _sc_gather_combine(a_t, b_t, g_t, idx_img, idx_subj, idx_obj):
    mesh = plsc.VectorSubcoreMesh(core_axis_name="c", subcore_axis_name="s")
    k = functools.partial(
        pl.kernel,
        mesh=mesh,
        out_type=jax.ShapeDtypeStruct((NREL_PAD, CP), jnp.float32),
        scratch_types=[
            pltpu.VMEM((PER_W,), jnp.int32),
            pltpu.VMEM((PER_W,), jnp.int32),
            pltpu.VMEM((PER_W,), jnp.int32),
            pltpu.VMEM((CH, CP), jnp.float32),
            pltpu.VMEM((CH, CP), jnp.float32),
            pltpu.SemaphoreType.DMA,
            pltpu.SemaphoreType.DMA,
            pltpu.SemaphoreType.DMA,
            pltpu.SemaphoreType.DMA,
            pltpu.SemaphoreType.DMA,
            pltpu.SemaphoreType.DMA,
            pltpu.SemaphoreType.DMA,
        ],
    )(_sc_body)
    return k(a_t, b_t, g_t, idx_img, idx_subj, idx_obj)


def kernel(obj_fmaps, obj_logits, vr, rel_inds, global_features,
           W_proj, b_proj, W_vr, b_vr):
    wt = W_vr.T
    pad_c = lambda w, c: jnp.pad(w, ((0, 0), (0, c - NCLS)))
    w1t = pad_c(wt[:D_OBJ], CP)
    w2t = pad_c(wt[D_OBJ:2 * D_OBJ], CP)
    wv = pad_c(wt[2 * D_OBJ:2 * D_OBJ + REL_OUT], VCP)
    wg = pad_c(wt[2 * D_OBJ + REL_OUT:], CP)
    w_ab = jnp.concatenate([w1t, w2t], axis=1)
    wpt = W_proj.T
    bp = b_proj[None, :]
    bv = pad_c(b_vr[None, :], VCP)

    a_t, b_t, g_t = _make_tables(obj_fmaps, global_features, w_ab, wg)
    r_t = _make_vrterm(vr, wpt, bp, wv, bv)
    preds = _make_argmax(obj_logits)

    pad_n = NREL_PAD - N_REL
    idx_img = jnp.pad(rel_inds[:, 0], (0, pad_n))
    idx_subj = jnp.pad(rel_inds[:, 1], (0, pad_n))
    idx_obj = jnp.pad(rel_inds[:, 2], (0, pad_n))
    s = _sc_gather_combine(a_t, b_t, g_t, idx_img, idx_subj, idx_obj)

    rel_dists = _make_combine(s, r_t)
    return (obj_logits, preds.reshape(-1), rel_dists)
